# baseline (speedup 1.0000x reference)
import jax
import jax.numpy as jnp
from jax import lax
from jax.experimental import pallas as pl
from jax.experimental.pallas import tpu as pltpu
from jax.experimental.pallas import tpu_sc as plsc

H = 1024
HW = H // 2
N = 4 * 8192
NC, NS, LANES = 2, 16, 16
NW = NC * NS
TPW = N // NW
T = 8
G = 4 * T
NCHUNK = TPW // T
PAIRS = NCHUNK // 2
WPT = HW // LANES

OFF_PCT = 512
OFF_FRAC = 616
NROWS_PAD = 1624


def _body(tab_h, pos_h, exp_h, pct_h, frac_h, d_h, out_h, dum_h,
          pe_i, pct_i, fb_i, d_v, gidx,
          buf_a, buf_b, o_a, o_b, semg_a, semg_b, semo_a, semo_b):
  wid = lax.axis_index("s") * NC + lax.axis_index("c")
  wbase = wid * TPW

  pltpu.sync_copy(pos_h.at[pl.ds(wbase, TPW)], pe_i)
  pltpu.sync_copy(exp_h.at[pl.ds(wbase, TPW)], pct_i)
  pltpu.sync_copy(frac_h.at[pl.ds(wbase, TPW)], fb_i)
  pltpu.sync_copy(d_h.at[pl.ds(wbase, TPW)], d_v.at[pl.ds(0, TPW)])

  @plsc.parallel_loop(0, TPW // LANES, unroll=4)
  def _(i):
    sl = pl.ds(i * LANES, LANES)
    pe_i[sl] = pe_i[sl] * 256 + pct_i[sl]
  pltpu.sync_copy(pct_h.at[pl.ds(wbase, TPW)], pct_i)

  lanes = lax.iota(jnp.int32, LANES)
  dest0 = jnp.where(lanes < T, 0, G) + (lanes & (T - 1))

  @plsc.parallel_loop(0, PAIRS, unroll=2)
  def _(p):
    tb = p * 2 * T
    dest = dest0 + p * (2 * G)
    pe16 = pe_i[pl.ds(tb, LANES)]
    pc16 = pct_i[pl.ds(tb, LANES)] + OFF_PCT
    fb16 = fb_i[pl.ds(tb, LANES)]
    plsc.store_scatter(gidx, [dest], pe16)
    plsc.store_scatter(gidx, [dest + T], pc16)
    plsc.store_scatter(gidx, [dest + 2 * T], fb16 + OFF_FRAC)
    plsc.store_scatter(gidx, [dest + 3 * T], fb16 + (OFF_FRAC + 1))

  def fire_gather(c, buf, sem):
    return pltpu.async_copy(tab_h.at[gidx.at[pl.ds(c * G, G)]], buf, sem)

  def drain_gather(buf, sem):
    pltpu.make_async_copy(tab_h.at[gidx.at[pl.ds(0, G)]], buf, sem).wait()

  def fire_scatter(c, o, sem):
    return pltpu.async_copy(o, out_h.at[pl.ds(wbase + c * T, T)], sem)

  def drain_scatter(o, sem):
    pltpu.make_async_copy(o, out_h.at[pl.ds(wbase, T)], sem).wait()

  fire_gather(0, buf_a, semg_a)
  fire_gather(1, buf_b, semg_b)
  pltpu.async_copy(o_a, dum_h.at[pl.ds(0, T)], semo_a)
  pltpu.async_copy(o_b, dum_h.at[pl.ds(T, T)], semo_b)

  def compute(c, buf, o):
    cb = c * T
    d16 = d_v[pl.ds(cb, LANES)]

    for t in range(T):
      dsp = jnp.broadcast_to(d16[t], (LANES,))
      one_m = 1.0 - dsp
      d32 = plsc.pack(dsp, dsp, format=plsc.PackFormat.INTERLEAVED)
      om32 = plsc.pack(one_m, one_m, format=plsc.PackFormat.INTERLEAVED)

      @plsc.parallel_loop(0, WPT, unroll=16)
      def _(w, t=t, d32=d32, om32=om32):
        slw = pl.ds(w * LANES, LANES)
        pe = plsc.bitcast(buf[t, slw], jnp.bfloat16)
        pc = plsc.bitcast(buf[T + t, slw], jnp.bfloat16)
        lo = plsc.bitcast(buf[2 * T + t, slw], jnp.bfloat16)
        hi = plsc.bitcast(buf[3 * T + t, slw], jnp.bfloat16)
        acc = pe + pc + om32 * lo + d32 * hi
        v = plsc.bitcast(acc, jnp.int32)
        ev = plsc.bitcast(lax.shift_left(v, 16), jnp.float32)
        od = plsc.bitcast(lax.bitwise_and(v, jnp.int32(-65536)), jnp.float32)
        col = w * LANES
        o[t, pl.ds(col, LANES)] = ev
        o[t, pl.ds(col + HW, LANES)] = od

  def pair_body(k, _):
    ca = 2 * k
    cb_ = 2 * k + 1
    drain_gather(buf_a, semg_a)
    drain_scatter(o_a, semo_a)
    compute(ca, buf_a, o_a)
    fire_scatter(ca, o_a, semo_a)
    fire_gather(jnp.minimum(ca + 2, NCHUNK - 2), buf_a, semg_a)
    drain_gather(buf_b, semg_b)
    drain_scatter(o_b, semo_b)
    compute(cb_, buf_b, o_b)
    fire_scatter(cb_, o_b, semo_b)
    fire_gather(jnp.minimum(cb_ + 2, NCHUNK - 1), buf_b, semg_b)
    return 0

  lax.fori_loop(0, PAIRS, pair_body, 0)

  drain_gather(buf_a, semg_a)
  drain_gather(buf_b, semg_b)
  drain_scatter(o_a, semo_a)
  drain_scatter(o_b, semo_b)


def _run(tab, pos, exp, pct, frac, delta):
  mesh = plsc.VectorSubcoreMesh(core_axis_name="c", subcore_axis_name="s")
  fn = pl.kernel(
      _body,
      out_type=(jax.ShapeDtypeStruct((N, H), jnp.float32),
                jax.ShapeDtypeStruct((2 * T, H), jnp.float32)),
      mesh=mesh,
      compiler_params=pltpu.CompilerParams(needs_layout_passes=False),
      scratch_types=[
          pltpu.VMEM((TPW,), jnp.int32),
          pltpu.VMEM((TPW,), jnp.int32),
          pltpu.VMEM((TPW,), jnp.int32),
          pltpu.VMEM((TPW + LANES,), jnp.float32),
          pltpu.VMEM((NCHUNK * G,), jnp.int32),
          pltpu.VMEM((G, HW), jnp.int32),
          pltpu.VMEM((G, HW), jnp.int32),
          pltpu.VMEM((T, H), jnp.float32),
          pltpu.VMEM((T, H), jnp.float32),
          pltpu.SemaphoreType.DMA,
          pltpu.SemaphoreType.DMA,
          pltpu.SemaphoreType.DMA,
          pltpu.SemaphoreType.DMA,
      ],
  )
  out, _ = fn(tab, pos, exp, pct, frac, delta)
  return out


def _prep_table(W_pos, W_exp, W_pct, W_frac):
  def pack(x):
    xb = x.astype(jnp.bfloat16)
    a = lax.bitcast_convert_type(xb[..., :HW], jnp.uint16).astype(jnp.uint32)
    b = lax.bitcast_convert_type(xb[..., HW:], jnp.uint16).astype(jnp.uint32)
    return lax.bitcast_convert_type(a | (b << 16), jnp.int32)

  def body(p_ref, e_ref, c_ref, f_ref, o_ref):
    pe = p_ref[...][:, None, :] + e_ref[...][None, :, :]
    o_ref[0:512] = pack(pe.reshape(512, H))
    o_ref[OFF_PCT:OFF_PCT + 102] = pack(c_ref[...])
    o_ref[OFF_FRAC:OFF_FRAC + 1001] = pack(f_ref[...])

  return pl.pallas_call(
      body,
      out_shape=jax.ShapeDtypeStruct((NROWS_PAD, HW), jnp.int32),
  )(W_pos, W_exp, W_pct, W_frac)


def kernel(is_positive, exponent, fraction_bin, delta, percentile_values,
           W_pos, W_exp, W_frac, W_pct):
  B, L = is_positive.shape
  tab_p = _prep_table(W_pos, W_exp, W_pct, W_frac)
  pos = is_positive.astype(jnp.int32).reshape(N)
  exp = exponent.astype(jnp.int32).reshape(N)
  pct = percentile_values.astype(jnp.int32).reshape(N)
  frac = fraction_bin.astype(jnp.int32).reshape(N)
  d = delta.astype(jnp.float32).reshape(N)
  out = _run(tab_p, pos, exp, pct, frac, d)
  return out.reshape(B, L, H)


# speedup vs baseline: 1.0033x; 1.0033x over previous
import jax
import jax.numpy as jnp
from jax import lax
from jax.experimental import pallas as pl
from jax.experimental.pallas import tpu as pltpu
from jax.experimental.pallas import tpu_sc as plsc

H = 1024
HW = H // 2
N = 4 * 8192
NC, NS, LANES = 2, 16, 16
NW = NC * NS
TPW = N // NW
T = 8
G = 4 * T
NCHUNK = TPW // T
PAIRS = NCHUNK // 2
WPT = HW // LANES

OFF_PCT = 512
OFF_FRAC = 616
NROWS_PAD = 1624


def _body(tab_h, pos_h, exp_h, pct_h, frac_h, d_h, out_h, dum_h,
          pe_i, pct_i, fb_i, d_v, gidx,
          buf_a, buf_b, o_a, o_b, semg_a, semg_b, semo_a, semo_b):
  wid = lax.axis_index("s") * NC + lax.axis_index("c")
  wbase = wid * TPW

  pltpu.sync_copy(pos_h.at[pl.ds(wbase, TPW)], pe_i)
  pltpu.sync_copy(exp_h.at[pl.ds(wbase, TPW)], pct_i)
  pltpu.sync_copy(frac_h.at[pl.ds(wbase, TPW)], fb_i)
  pltpu.sync_copy(d_h.at[pl.ds(wbase, TPW)], d_v.at[pl.ds(0, TPW)])

  @plsc.parallel_loop(0, TPW // LANES, unroll=4)
  def _(i):
    sl = pl.ds(i * LANES, LANES)
    pe_i[sl] = pe_i[sl] * 256 + pct_i[sl]
  pltpu.sync_copy(pct_h.at[pl.ds(wbase, TPW)], pct_i)

  lanes = lax.iota(jnp.int32, LANES)
  dest0 = jnp.where(lanes < T, 0, G) + (lanes & (T - 1))

  @plsc.parallel_loop(0, PAIRS, unroll=2)
  def _(p):
    tb = p * 2 * T
    dest = dest0 + p * (2 * G)
    pe16 = pe_i[pl.ds(tb, LANES)]
    pc16 = pct_i[pl.ds(tb, LANES)] + OFF_PCT
    fb16 = fb_i[pl.ds(tb, LANES)]
    plsc.store_scatter(gidx, [dest], pe16)
    plsc.store_scatter(gidx, [dest + T], pc16)
    plsc.store_scatter(gidx, [dest + 2 * T], fb16 + OFF_FRAC)
    plsc.store_scatter(gidx, [dest + 3 * T], fb16 + (OFF_FRAC + 1))

  def fire_gather(c, buf, sem):
    return pltpu.async_copy(tab_h.at[gidx.at[pl.ds(c * G, G)]], buf, sem)

  def drain_gather(buf, sem):
    pltpu.make_async_copy(tab_h.at[gidx.at[pl.ds(0, G)]], buf, sem).wait()

  def fire_scatter(c, o, sem):
    return pltpu.async_copy(o, out_h.at[pl.ds(wbase + c * T, T)], sem)

  def drain_scatter(o, sem):
    pltpu.make_async_copy(o, out_h.at[pl.ds(wbase, T)], sem).wait()

  fire_gather(0, buf_a, semg_a)
  fire_gather(1, buf_b, semg_b)
  pltpu.async_copy(o_a, dum_h.at[pl.ds(0, T)], semo_a)
  pltpu.async_copy(o_b, dum_h.at[pl.ds(T, T)], semo_b)

  def compute(c, buf, o):
    cb = c * T
    d16 = d_v[pl.ds(cb, LANES)]

    for t in range(T):
      dsp = jnp.broadcast_to(d16[t], (LANES,))
      one_m = 1.0 - dsp
      d32 = plsc.pack(dsp, dsp, format=plsc.PackFormat.INTERLEAVED)
      om32 = plsc.pack(one_m, one_m, format=plsc.PackFormat.INTERLEAVED)

      @plsc.parallel_loop(0, WPT, unroll=4)
      def _(w, t=t, d32=d32, om32=om32):
        slw = pl.ds(w * LANES, LANES)
        pe = plsc.bitcast(buf[t, slw], jnp.bfloat16)
        pc = plsc.bitcast(buf[T + t, slw], jnp.bfloat16)
        lo = plsc.bitcast(buf[2 * T + t, slw], jnp.bfloat16)
        hi = plsc.bitcast(buf[3 * T + t, slw], jnp.bfloat16)
        acc = pe + pc + om32 * lo + d32 * hi
        v = plsc.bitcast(acc, jnp.int32)
        ev = plsc.bitcast(lax.shift_left(v, 16), jnp.float32)
        od = plsc.bitcast(lax.bitwise_and(v, jnp.int32(-65536)), jnp.float32)
        col = w * LANES
        o[t, pl.ds(col, LANES)] = ev
        o[t, pl.ds(col + HW, LANES)] = od

  def pair_body(k, _):
    ca = 2 * k
    cb_ = 2 * k + 1
    drain_gather(buf_a, semg_a)
    drain_scatter(o_a, semo_a)
    compute(ca, buf_a, o_a)
    fire_scatter(ca, o_a, semo_a)
    fire_gather(jnp.minimum(ca + 2, NCHUNK - 2), buf_a, semg_a)
    drain_gather(buf_b, semg_b)
    drain_scatter(o_b, semo_b)
    compute(cb_, buf_b, o_b)
    fire_scatter(cb_, o_b, semo_b)
    fire_gather(jnp.minimum(cb_ + 2, NCHUNK - 1), buf_b, semg_b)
    return 0

  lax.fori_loop(0, PAIRS, pair_body, 0)

  drain_gather(buf_a, semg_a)
  drain_gather(buf_b, semg_b)
  drain_scatter(o_a, semo_a)
  drain_scatter(o_b, semo_b)


def _run(tab, pos, exp, pct, frac, delta):
  mesh = plsc.VectorSubcoreMesh(core_axis_name="c", subcore_axis_name="s")
  fn = pl.kernel(
      _body,
      out_type=(jax.ShapeDtypeStruct((N, H), jnp.float32),
                jax.ShapeDtypeStruct((2 * T, H), jnp.float32)),
      mesh=mesh,
      compiler_params=pltpu.CompilerParams(needs_layout_passes=False),
      scratch_types=[
          pltpu.VMEM((TPW,), jnp.int32),
          pltpu.VMEM((TPW,), jnp.int32),
          pltpu.VMEM((TPW,), jnp.int32),
          pltpu.VMEM((TPW + LANES,), jnp.float32),
          pltpu.VMEM((NCHUNK * G,), jnp.int32),
          pltpu.VMEM((G, HW), jnp.int32),
          pltpu.VMEM((G, HW), jnp.int32),
          pltpu.VMEM((T, H), jnp.float32),
          pltpu.VMEM((T, H), jnp.float32),
          pltpu.SemaphoreType.DMA,
          pltpu.SemaphoreType.DMA,
          pltpu.SemaphoreType.DMA,
          pltpu.SemaphoreType.DMA,
      ],
  )
  out, _ = fn(tab, pos, exp, pct, frac, delta)
  return out


def _prep_table(W_pos, W_exp, W_pct, W_frac):
  def pack(x):
    xb = x.astype(jnp.bfloat16)
    a = lax.bitcast_convert_type(xb[..., :HW], jnp.uint16).astype(jnp.uint32)
    b = lax.bitcast_convert_type(xb[..., HW:], jnp.uint16).astype(jnp.uint32)
    return lax.bitcast_convert_type(a | (b << 16), jnp.int32)

  def body(p_ref, e_ref, c_ref, f_ref, o_ref):
    pe = p_ref[...][:, None, :] + e_ref[...][None, :, :]
    o_ref[0:512] = pack(pe.reshape(512, H))
    o_ref[OFF_PCT:OFF_PCT + 102] = pack(c_ref[...])
    o_ref[OFF_FRAC:OFF_FRAC + 1001] = pack(f_ref[...])

  return pl.pallas_call(
      body,
      out_shape=jax.ShapeDtypeStruct((NROWS_PAD, HW), jnp.int32),
  )(W_pos, W_exp, W_pct, W_frac)


def kernel(is_positive, exponent, fraction_bin, delta, percentile_values,
           W_pos, W_exp, W_frac, W_pct):
  B, L = is_positive.shape
  tab_p = _prep_table(W_pos, W_exp, W_pct, W_frac)
  pos = is_positive.astype(jnp.int32).reshape(N)
  exp = exponent.astype(jnp.int32).reshape(N)
  pct = percentile_values.astype(jnp.int32).reshape(N)
  frac = fraction_bin.astype(jnp.int32).reshape(N)
  d = delta.astype(jnp.float32).reshape(N)
  out = _run(tab_p, pos, exp, pct, frac, d)
  return out.reshape(B, L, H)


# speedup vs baseline: 1.0073x; 1.0040x over previous
import jax
import jax.numpy as jnp
from jax import lax
from jax.experimental import pallas as pl
from jax.experimental.pallas import tpu as pltpu
from jax.experimental.pallas import tpu_sc as plsc

H = 1024
HW = H // 2
N = 4 * 8192
NC, NS, LANES = 2, 16, 16
NW = NC * NS
TPW = N // NW
T = 8
G = 4 * T
NCHUNK = TPW // T
PAIRS = NCHUNK // 2
WPT = HW // LANES

OFF_PCT = 512
OFF_FRAC = 616
NROWS_PAD = 1624


def _body(tab_h, pos_h, exp_h, pct_h, frac_h, d_h, out_h, dum_h,
          pe_i, pct_i, fb_i, d_v, gidx,
          buf_a, buf_b, o_a, o_b, semg_a, semg_b, semo_a, semo_b):
  wid = lax.axis_index("s") * NC + lax.axis_index("c")
  wbase = wid * TPW

  pltpu.sync_copy(pos_h.at[pl.ds(wbase, TPW)], pe_i)
  pltpu.sync_copy(exp_h.at[pl.ds(wbase, TPW)], pct_i)
  pltpu.sync_copy(frac_h.at[pl.ds(wbase, TPW)], fb_i)
  pltpu.sync_copy(d_h.at[pl.ds(wbase, TPW)], d_v.at[pl.ds(0, TPW)])

  @plsc.parallel_loop(0, TPW // LANES, unroll=4)
  def _(i):
    sl = pl.ds(i * LANES, LANES)
    pe_i[sl] = pe_i[sl] * 256 + pct_i[sl]
  pltpu.sync_copy(pct_h.at[pl.ds(wbase, TPW)], pct_i)

  lanes = lax.iota(jnp.int32, LANES)
  dest0 = jnp.where(lanes < T, 0, G) + (lanes & (T - 1))

  @plsc.parallel_loop(0, PAIRS, unroll=2)
  def _(p):
    tb = p * 2 * T
    dest = dest0 + p * (2 * G)
    pe16 = pe_i[pl.ds(tb, LANES)]
    pc16 = pct_i[pl.ds(tb, LANES)] + OFF_PCT
    fb16 = fb_i[pl.ds(tb, LANES)]
    plsc.store_scatter(gidx, [dest], pe16)
    plsc.store_scatter(gidx, [dest + T], pc16)
    plsc.store_scatter(gidx, [dest + 2 * T], fb16 + OFF_FRAC)
    plsc.store_scatter(gidx, [dest + 3 * T], fb16 + (OFF_FRAC + 1))

  def fire_gather(c, buf, sem):
    return pltpu.async_copy(tab_h.at[gidx.at[pl.ds(c * G, G)]], buf, sem)

  def drain_gather(buf, sem):
    pltpu.make_async_copy(tab_h.at[gidx.at[pl.ds(0, G)]], buf, sem).wait()

  def fire_scatter(c, o, sem):
    return pltpu.async_copy(o, out_h.at[pl.ds(wbase + c * T, T)], sem)

  def drain_scatter(o, sem):
    pltpu.make_async_copy(o, out_h.at[pl.ds(wbase, T)], sem).wait()

  fire_gather(0, buf_a, semg_a)
  fire_gather(1, buf_b, semg_b)
  pltpu.async_copy(o_a, dum_h.at[pl.ds(0, T)], semo_a)
  pltpu.async_copy(o_b, dum_h.at[pl.ds(T, T)], semo_b)

  def compute(c, buf, o):
    cb = c * T
    d16 = d_v[pl.ds(cb, LANES)]

    for t in range(T):
      dsp = jnp.broadcast_to(d16[t], (LANES,))
      one_m = 1.0 - dsp
      d32 = plsc.pack(dsp, dsp, format=plsc.PackFormat.INTERLEAVED)
      om32 = plsc.pack(one_m, one_m, format=plsc.PackFormat.INTERLEAVED)

      @plsc.parallel_loop(0, WPT, unroll=8)
      def _(w, t=t, d32=d32, om32=om32):
        slw = pl.ds(w * LANES, LANES)
        pe = plsc.bitcast(buf[t, slw], jnp.bfloat16)
        pc = plsc.bitcast(buf[T + t, slw], jnp.bfloat16)
        lo = plsc.bitcast(buf[2 * T + t, slw], jnp.bfloat16)
        hi = plsc.bitcast(buf[3 * T + t, slw], jnp.bfloat16)
        acc = pe + pc + om32 * lo + d32 * hi
        v = plsc.bitcast(acc, jnp.int32)
        ev = plsc.bitcast(lax.shift_left(v, 16), jnp.float32)
        od = plsc.bitcast(lax.bitwise_and(v, jnp.int32(-65536)), jnp.float32)
        col = w * LANES
        o[t, pl.ds(col, LANES)] = ev
        o[t, pl.ds(col + HW, LANES)] = od

  def pair_body(k, _):
    ca = 2 * k
    cb_ = 2 * k + 1
    drain_gather(buf_a, semg_a)
    drain_scatter(o_a, semo_a)
    compute(ca, buf_a, o_a)
    fire_scatter(ca, o_a, semo_a)
    fire_gather(jnp.minimum(ca + 2, NCHUNK - 2), buf_a, semg_a)
    drain_gather(buf_b, semg_b)
    drain_scatter(o_b, semo_b)
    compute(cb_, buf_b, o_b)
    fire_scatter(cb_, o_b, semo_b)
    fire_gather(jnp.minimum(cb_ + 2, NCHUNK - 1), buf_b, semg_b)
    return 0

  lax.fori_loop(0, PAIRS, pair_body, 0)

  drain_gather(buf_a, semg_a)
  drain_gather(buf_b, semg_b)
  drain_scatter(o_a, semo_a)
  drain_scatter(o_b, semo_b)


def _run(tab, pos, exp, pct, frac, delta):
  mesh = plsc.VectorSubcoreMesh(core_axis_name="c", subcore_axis_name="s")
  fn = pl.kernel(
      _body,
      out_type=(jax.ShapeDtypeStruct((N, H), jnp.float32),
                jax.ShapeDtypeStruct((2 * T, H), jnp.float32)),
      mesh=mesh,
      compiler_params=pltpu.CompilerParams(needs_layout_passes=False),
      scratch_types=[
          pltpu.VMEM((TPW,), jnp.int32),
          pltpu.VMEM((TPW,), jnp.int32),
          pltpu.VMEM((TPW,), jnp.int32),
          pltpu.VMEM((TPW + LANES,), jnp.float32),
          pltpu.VMEM((NCHUNK * G,), jnp.int32),
          pltpu.VMEM((G, HW), jnp.int32),
          pltpu.VMEM((G, HW), jnp.int32),
          pltpu.VMEM((T, H), jnp.float32),
          pltpu.VMEM((T, H), jnp.float32),
          pltpu.SemaphoreType.DMA,
          pltpu.SemaphoreType.DMA,
          pltpu.SemaphoreType.DMA,
          pltpu.SemaphoreType.DMA,
      ],
  )
  out, _ = fn(tab, pos, exp, pct, frac, delta)
  return out


def _prep_table(W_pos, W_exp, W_pct, W_frac):
  def pack(x):
    xb = x.astype(jnp.bfloat16)
    a = lax.bitcast_convert_type(xb[..., :HW], jnp.uint16).astype(jnp.uint32)
    b = lax.bitcast_convert_type(xb[..., HW:], jnp.uint16).astype(jnp.uint32)
    return lax.bitcast_convert_type(a | (b << 16), jnp.int32)

  def body(p_ref, e_ref, c_ref, f_ref, o_ref):
    pe = p_ref[...][:, None, :] + e_ref[...][None, :, :]
    o_ref[0:512] = pack(pe.reshape(512, H))
    o_ref[OFF_PCT:OFF_PCT + 102] = pack(c_ref[...])
    o_ref[OFF_FRAC:OFF_FRAC + 1001] = pack(f_ref[...])

  return pl.pallas_call(
      body,
      out_shape=jax.ShapeDtypeStruct((NROWS_PAD, HW), jnp.int32),
  )(W_pos, W_exp, W_pct, W_frac)


def kernel(is_positive, exponent, fraction_bin, delta, percentile_values,
           W_pos, W_exp, W_frac, W_pct):
  B, L = is_positive.shape
  tab_p = _prep_table(W_pos, W_exp, W_pct, W_frac)
  pos = is_positive.astype(jnp.int32).reshape(N)
  exp = exponent.astype(jnp.int32).reshape(N)
  pct = percentile_values.astype(jnp.int32).reshape(N)
  frac = fraction_bin.astype(jnp.int32).reshape(N)
  d = delta.astype(jnp.float32).reshape(N)
  out = _run(tab_p, pos, exp, pct, frac, d)
  return out.reshape(B, L, H)
